# baseline (speedup 1.0000x reference)
import functools

import jax
import jax.numpy as jnp
from jax import lax
from jax.experimental import pallas as pl
from jax.experimental.pallas import tpu as pltpu
from jax.experimental.pallas import tpu_sc as plsc

NUM_ITEM = 1000000
HIDDEN = 64
BATCH = 16384
HIST = 20

_L = 16
_NC = 2
_NS = 16
_NW = _NC * _NS
_B = BATCH * HIST
_BPW = _B // _NW
_CH = 128
_NCHUNK = _BPW // _CH
_TP = 129


def _compute_chunk(e_v, s_v, t_v):
    lane = lax.iota(jnp.int32, _L)

    def row_body(r, carry):
        for j in range(HIDDEN // _L):
            sl = pl.ds(j * _L, _L)
            v = e_v[r, sl]
            t = s_v[r, sl]
            sig = 1.0 / (1.0 + jnp.exp(-t))
            res = v - jnp.minimum(jnp.maximum(v, -sig), sig)
            plsc.store_scatter(t_v, [lane + j * _L, lane * 0 + r], res)
        return carry

    lax.fori_loop(0, _CH, row_body, 0, unroll=False)


@functools.partial(
    pl.kernel,
    out_type=jax.ShapeDtypeStruct((HIST, HIDDEN, BATCH), jnp.float32),
    mesh=plsc.VectorSubcoreMesh(core_axis_name="c", subcore_axis_name="s"),
    compiler_params=pltpu.CompilerParams(
        use_tc_tiling_on_sc=False, needs_layout_passes=False),
    scratch_types=[
        pltpu.VMEM((_CH,), jnp.int32),
        pltpu.VMEM((_CH,), jnp.int32),
        pltpu.VMEM((_CH, HIDDEN), jnp.float32),
        pltpu.VMEM((_CH, HIDDEN), jnp.float32),
        pltpu.VMEM((_CH, HIDDEN), jnp.float32),
        pltpu.VMEM((_CH, HIDDEN), jnp.float32),
        pltpu.VMEM((HIDDEN, _TP), jnp.float32),
        pltpu.VMEM((HIDDEN, _TP), jnp.float32),
        pltpu.SemaphoreType.DMA,
        pltpu.SemaphoreType.DMA,
        pltpu.SemaphoreType.DMA,
        pltpu.SemaphoreType.DMA,
        pltpu.SemaphoreType.DMA,
        pltpu.SemaphoreType.DMA,
        pltpu.SemaphoreType.DMA,
    ],
)
def _sc_lookup(idx_hbm, emb_hbm, s_hbm, out_hbm,
               idx0, idx1, e0, e1, s0, s1, t0, t1,
               sem_i, sem_e0, sem_e1, sem_s0, sem_s1, sem_o0, sem_o1):
    wid = lax.axis_index("s") * _NC + lax.axis_index("c")
    base = wid * _BPW
    idx_b = (idx0, idx1)
    e_b = (e0, e1)
    s_b = (s0, s1)
    t_b = (t0, t1)
    sem_e = (sem_e0, sem_e1)
    sem_s = (sem_s0, sem_s1)
    sem_o = (sem_o0, sem_o1)

    def out_ref(c):
        p_lo = base + c * _CH
        return out_hbm.at[p_lo // BATCH, :, pl.ds(p_lo % BATCH, _CH)]

    pltpu.sync_copy(idx_hbm.at[pl.ds(base, _CH)], idx0)
    pltpu.async_copy(emb_hbm.at[idx0], e0, sem_e0)
    pltpu.async_copy(s_hbm.at[idx0], s0, sem_s0)
    pltpu.async_copy(idx_hbm.at[pl.ds(base + _CH, _CH)], idx1, sem_i).wait()
    pltpu.async_copy(emb_hbm.at[idx1], e1, sem_e1)
    pltpu.async_copy(s_hbm.at[idx1], s1, sem_s1)

    def pair_body(g, carry):
        for b in range(2):
            c = g * 2 + b
            pltpu.make_async_copy(emb_hbm.at[idx_b[b]], e_b[b], sem_e[b]).wait()
            pltpu.make_async_copy(s_hbm.at[idx_b[b]], s_b[b], sem_s[b]).wait()

            @pl.when(c >= 2)
            def _():
                pltpu.make_async_copy(
                    t_b[b].at[:, pl.ds(0, _CH)], out_ref(c - 2), sem_o[b]).wait()

            _compute_chunk(e_b[b], s_b[b], t_b[b])
            pltpu.async_copy(t_b[b].at[:, pl.ds(0, _CH)], out_ref(c), sem_o[b])

            @pl.when(c + 2 < _NCHUNK)
            def _():
                off_n = base + (c + 2) * _CH
                pltpu.async_copy(idx_hbm.at[pl.ds(off_n, _CH)], idx_b[b], sem_i).wait()
                pltpu.async_copy(emb_hbm.at[idx_b[b]], e_b[b], sem_e[b])
                pltpu.async_copy(s_hbm.at[idx_b[b]], s_b[b], sem_s[b])
        return carry

    lax.fori_loop(0, _NCHUNK // 2, pair_body, 0, unroll=False)

    for b in range(2):
        pltpu.make_async_copy(
            t_b[b].at[:, pl.ds(0, _CH)], out_ref(_NCHUNK - 2 + b),
            sem_o[b]).wait()


def kernel(x, emb_weight, s):
    idx = x.T.reshape(-1).astype(jnp.int32)
    out = _sc_lookup(idx, emb_weight, s)
    return out.transpose(2, 0, 1)


# speedup vs baseline: 1.4870x; 1.4870x over previous
import functools

import jax
import jax.numpy as jnp
from jax import lax
from jax.experimental import pallas as pl
from jax.experimental.pallas import tpu as pltpu
from jax.experimental.pallas import tpu_sc as plsc

NUM_ITEM = 1000000
HIDDEN = 64
BATCH = 16384
HIST = 20

_L = 16
_NC = 2
_NS = 16
_NW = _NC * _NS
_B = BATCH * HIST
_BPW = _B // _NW
_CH = 128
_NCHUNK = _BPW // _CH


def _soft_threshold_chunk(e_v, s_v):
    def row_body(r, carry):
        for j in range(HIDDEN // _L):
            sl = pl.ds(j * _L, _L)
            v = e_v[r, sl]
            t = s_v[r, sl]
            sig = 1.0 / (1.0 + jnp.exp(-t))
            e_v[r, sl] = v - jnp.minimum(jnp.maximum(v, -sig), sig)
        return carry

    lax.fori_loop(0, _CH, row_body, 0, unroll=False)


@functools.partial(
    pl.kernel,
    out_type=jax.ShapeDtypeStruct((_B, HIDDEN), jnp.float32),
    mesh=plsc.VectorSubcoreMesh(core_axis_name="c", subcore_axis_name="s"),
    compiler_params=pltpu.CompilerParams(use_tc_tiling_on_sc=False),
    scratch_types=[
        pltpu.VMEM((_CH,), jnp.int32),
        pltpu.VMEM((_CH,), jnp.int32),
        pltpu.VMEM((_CH, HIDDEN), jnp.float32),
        pltpu.VMEM((_CH, HIDDEN), jnp.float32),
        pltpu.VMEM((_CH, HIDDEN), jnp.float32),
        pltpu.VMEM((_CH, HIDDEN), jnp.float32),
        pltpu.SemaphoreType.DMA,
        pltpu.SemaphoreType.DMA,
        pltpu.SemaphoreType.DMA,
        pltpu.SemaphoreType.DMA,
        pltpu.SemaphoreType.DMA,
    ],
)
def _sc_lookup(idx_hbm, emb_hbm, s_hbm, out_hbm,
               idx0, idx1, e0, e1, s0, s1,
               sem_i, sem_e0, sem_e1, sem_s0, sem_s1):
    wid = lax.axis_index("s") * _NC + lax.axis_index("c")
    base = wid * _BPW
    idx_b = (idx0, idx1)
    e_b = (e0, e1)
    s_b = (s0, s1)
    sem_e = (sem_e0, sem_e1)
    sem_s = (sem_s0, sem_s1)

    pltpu.sync_copy(idx_hbm.at[pl.ds(base, _CH)], idx0)
    pltpu.async_copy(emb_hbm.at[idx0], e0, sem_e0)
    pltpu.async_copy(s_hbm.at[idx0], s0, sem_s0)
    pltpu.async_copy(idx_hbm.at[pl.ds(base + _CH, _CH)], idx1, sem_i).wait()
    pltpu.async_copy(emb_hbm.at[idx1], e1, sem_e1)
    pltpu.async_copy(s_hbm.at[idx1], s1, sem_s1)

    def pair_body(g, carry):
        for b in range(2):
            c = g * 2 + b
            pltpu.make_async_copy(emb_hbm.at[idx_b[b]], e_b[b], sem_e[b]).wait()
            pltpu.make_async_copy(s_hbm.at[idx_b[b]], s_b[b], sem_s[b]).wait()
            _soft_threshold_chunk(e_b[b], s_b[b])
            pltpu.sync_copy(e_b[b], out_hbm.at[pl.ds(base + c * _CH, _CH)])
            @pl.when(c + 2 < _NCHUNK)
            def _():
                off_n = base + (c + 2) * _CH
                pltpu.async_copy(idx_hbm.at[pl.ds(off_n, _CH)], idx_b[b], sem_i).wait()
                pltpu.async_copy(emb_hbm.at[idx_b[b]], e_b[b], sem_e[b])
                pltpu.async_copy(s_hbm.at[idx_b[b]], s_b[b], sem_s[b])
        return carry

    lax.fori_loop(0, _NCHUNK // 2, pair_body, 0, unroll=False)


def kernel(x, emb_weight, s):
    idx = x.T.reshape(-1).astype(jnp.int32)
    out = _sc_lookup(idx, emb_weight, s)
    return out.reshape(HIST, BATCH, HIDDEN).transpose(1, 0, 2)


# speedup vs baseline: 1.4880x; 1.0007x over previous
import functools

import jax
import jax.numpy as jnp
from jax import lax
from jax.experimental import pallas as pl
from jax.experimental.pallas import tpu as pltpu
from jax.experimental.pallas import tpu_sc as plsc

NUM_ITEM = 1000000
HIDDEN = 64
BATCH = 16384
HIST = 20

_L = 16
_NC = 2
_NS = 16
_NW = _NC * _NS
_B = BATCH * HIST
_BPW = _B // _NW
_CH = 128
_NCHUNK = _BPW // _CH


def _soft_threshold_chunk(e_v, s_v):
    def row_body(r, carry):
        for j in range(HIDDEN // _L):
            sl = pl.ds(j * _L, _L)
            v = e_v[r, sl]
            t = s_v[r, sl]
            sig = 1.0 / (1.0 + jnp.exp(-t))
            e_v[r, sl] = v - jnp.minimum(jnp.maximum(v, -sig), sig)
        return carry

    lax.fori_loop(0, _CH, row_body, 0, unroll=False)


@functools.partial(
    pl.kernel,
    out_type=jax.ShapeDtypeStruct((_B, HIDDEN), jnp.float32),
    mesh=plsc.VectorSubcoreMesh(core_axis_name="c", subcore_axis_name="s"),
    compiler_params=pltpu.CompilerParams(use_tc_tiling_on_sc=False),
    scratch_types=(
        [pltpu.VMEM((_CH,), jnp.int32)] * 4
        + [pltpu.VMEM((_CH, HIDDEN), jnp.float32)] * 8
        + [pltpu.SemaphoreType.DMA] * 9
    ),
)
def _sc_lookup(idx_hbm, emb_hbm, s_hbm, out_hbm,
               idx0, idx1, idx2, idx3, e0, e1, e2, e3, s0, s1, s2, s3,
               sem_i, sem_e0, sem_e1, sem_e2, sem_e3,
               sem_s0, sem_s1, sem_s2, sem_s3):
    wid = lax.axis_index("s") * _NC + lax.axis_index("c")
    base = wid * _BPW
    idx_b = (idx0, idx1, idx2, idx3)
    e_b = (e0, e1, e2, e3)
    s_b = (s0, s1, s2, s3)
    sem_e = (sem_e0, sem_e1, sem_e2, sem_e3)
    sem_s = (sem_s0, sem_s1, sem_s2, sem_s3)

    pltpu.sync_copy(idx_hbm.at[pl.ds(base, _CH)], idx0)
    pltpu.async_copy(emb_hbm.at[idx0], e0, sem_e0)
    pltpu.async_copy(s_hbm.at[idx0], s0, sem_s0)
    for b in range(1, 4):
        pltpu.async_copy(idx_hbm.at[pl.ds(base + b * _CH, _CH)], idx_b[b],
                         sem_e[b]).wait()
        pltpu.async_copy(emb_hbm.at[idx_b[b]], e_b[b], sem_e[b])
        pltpu.async_copy(s_hbm.at[idx_b[b]], s_b[b], sem_s[b])

    def quad_body(g, carry):
        for b in range(4):
            c = g * 4 + b
            pltpu.make_async_copy(emb_hbm.at[idx_b[b]], e_b[b], sem_e[b]).wait()
            pltpu.make_async_copy(s_hbm.at[idx_b[b]], s_b[b], sem_s[b]).wait()
            _soft_threshold_chunk(e_b[b], s_b[b])
            pltpu.sync_copy(e_b[b], out_hbm.at[pl.ds(base + c * _CH, _CH)])
            @pl.when(c + 4 < _NCHUNK)
            def _():
                off_n = base + (c + 4) * _CH
                pltpu.async_copy(idx_hbm.at[pl.ds(off_n, _CH)], idx_b[b],
                                 sem_e[b]).wait()
                pltpu.async_copy(emb_hbm.at[idx_b[b]], e_b[b], sem_e[b])
                pltpu.async_copy(s_hbm.at[idx_b[b]], s_b[b], sem_s[b])
        return carry

    lax.fori_loop(0, _NCHUNK // 4, quad_body, 0, unroll=False)


def kernel(x, emb_weight, s):
    idx = x.T.reshape(-1).astype(jnp.int32)
    out = _sc_lookup(idx, emb_weight, s)
    return out.reshape(HIST, BATCH, HIDDEN).transpose(1, 0, 2)
